# baseline (speedup 1.0000x reference)
import functools

import jax
import jax.numpy as jnp
from jax import lax
from jax.experimental import pallas as pl
from jax.experimental.pallas import tpu as pltpu
from jax.experimental.pallas import tpu_sc as plsc

_LANES = 16


def _bcast(v, t):
    dn = lax.GatherDimensionNumbers(
        offset_dims=(), collapsed_slice_dims=(0,), start_index_map=(0,))
    return lax.gather(v, jnp.full((_LANES, 1), t, jnp.int32), dn, (1,),
                      mode=lax.GatherScatterMode.PROMISE_IN_BOUNDS)


def _stage_a(cur_msg, WqT, Wk, bq2):
    B, D = cur_msg.shape
    BM = 256

    def body(msg_ref, WqT_ref, Wk_ref, bq2_ref, q2_ref, Wqk_s, bqk_s):
        @pl.when(pl.program_id(0) == 0)
        def _():
            Wqk_s[...] = jnp.dot(WqT_ref[...], Wk_ref[...],
                                 preferred_element_type=jnp.float32)
            bqk_s[...] = jnp.dot(bq2_ref[...], Wk_ref[...],
                                 preferred_element_type=jnp.float32)

        q2_ref[...] = jnp.dot(msg_ref[...], Wqk_s[...],
                              preferred_element_type=jnp.float32) + bqk_s[...]

    return pl.pallas_call(
        body,
        grid=(B // BM,),
        in_specs=[
            pl.BlockSpec((BM, D), lambda i: (i, 0)),
            pl.BlockSpec((D, D), lambda i: (0, 0)),
            pl.BlockSpec((D, D), lambda i: (0, 0)),
            pl.BlockSpec((1, D), lambda i: (0, 0)),
        ],
        out_specs=pl.BlockSpec((BM, D), lambda i: (i, 0)),
        out_shape=jax.ShapeDtypeStruct((B, D), jnp.float32),
        scratch_shapes=[pltpu.VMEM((D, D), jnp.float32),
                        pltpu.VMEM((1, D), jnp.float32)],
    )(cur_msg, WqT, Wk, bq2)


def _stage_b(idx, q2, bank2):
    B, D = q2.shape
    MD = bank2.shape[1]
    M = MD // D
    NW = 32
    bw = B // NW
    NB = 8
    nchunks = bw // NB
    scale = D ** -0.5
    DC = D // _LANES

    mesh = plsc.VectorSubcoreMesh(core_axis_name="c", subcore_axis_name="s")

    @functools.partial(
        pl.kernel, mesh=mesh,
        out_type=jax.ShapeDtypeStruct((B, D), jnp.float32),
        compiler_params=pltpu.CompilerParams(needs_layout_passes=False),
        scratch_types=[
            pltpu.VMEM((bw,), jnp.int32),
            pltpu.VMEM((NB, MD), jnp.float32),
            pltpu.VMEM((NB, D), jnp.float32),
            pltpu.VMEM((NB, D), jnp.float32),
            pltpu.SemaphoreType.DMA,
        ])
    def sc(idx_hbm, q2_hbm, bank_hbm, p_hbm, idx_v, rows_v, q2_v, p_v, sem):
        wid = lax.axis_index("s") * 2 + lax.axis_index("c")
        base = wid * bw
        pltpu.sync_copy(idx_hbm.at[pl.ds(base, bw)], idx_v)
        lane = lax.broadcasted_iota(jnp.int32, (_LANES,), 0)
        mbase = lane * D

        def chunk(c, carry):
            off = c * NB
            pltpu.async_copy(
                bank_hbm.at[idx_v.at[pl.ds(off, NB)]], rows_v, sem).wait()
            pltpu.sync_copy(q2_hbm.at[pl.ds(base + off, NB)], q2_v)
            for j in range(NB):
                jsplat = jnp.full((_LANES,), j, jnp.int32)

                def p1(dc, acc):
                    q2c = q2_v[j, pl.ds(dc * _LANES, _LANES)]
                    for t in range(_LANES):
                        qb = _bcast(q2c, t)
                        g = plsc.load_gather(
                            rows_v, [jsplat, mbase + (dc * _LANES + t)])
                        acc = acc + g * qb
                    return acc

                s = lax.fori_loop(0, DC, p1,
                                  jnp.zeros((_LANES,), jnp.float32)) * scale
                e = jnp.exp(s - jnp.max(s))
                a = e / jnp.sum(e)

                def p2(dc, carry2):
                    acc = jnp.zeros((_LANES,), jnp.float32)
                    for m in range(M):
                        acc = acc + (
                            rows_v[j, pl.ds(m * D + dc * _LANES, _LANES)]
                            * _bcast(a, m))
                    p_v[j, pl.ds(dc * _LANES, _LANES)] = acc
                    return carry2

                lax.fori_loop(0, DC, p2, 0)
            pltpu.sync_copy(p_v, p_hbm.at[pl.ds(base + off, NB)])
            return carry

        lax.fori_loop(0, nchunks, chunk, 0)

    return sc(idx, q2, bank2)


def _stage_c(p, cur_msg, WvT, WoT, bv2, bo2, g2, b2):
    B, D = cur_msg.shape
    BM = 256

    def body(p_ref, msg_ref, WvT_ref, WoT_ref, bv2_ref, bo2_ref,
             g_ref, b_ref, o_ref, Wvo_s, bvo_s):
        @pl.when(pl.program_id(0) == 0)
        def _():
            Wvo_s[...] = jnp.dot(WvT_ref[...], WoT_ref[...],
                                 preferred_element_type=jnp.float32)
            bvo_s[...] = jnp.dot(bv2_ref[...], WoT_ref[...],
                                 preferred_element_type=jnp.float32) + bo2_ref[...]

        x = msg_ref[...] + jnp.dot(p_ref[...], Wvo_s[...],
                                   preferred_element_type=jnp.float32) + bvo_s[...]
        mu = jnp.mean(x, axis=1, keepdims=True)
        xc = x - mu
        var = jnp.mean(xc * xc, axis=1, keepdims=True)
        o_ref[...] = xc * lax.rsqrt(var + 1e-5) * g_ref[...] + b_ref[...]

    full = lambda i: (0, 0)
    blk = lambda i: (i, 0)
    return pl.pallas_call(
        body,
        grid=(B // BM,),
        in_specs=[
            pl.BlockSpec((BM, D), blk),
            pl.BlockSpec((BM, D), blk),
            pl.BlockSpec((D, D), full),
            pl.BlockSpec((D, D), full),
            pl.BlockSpec((1, D), full),
            pl.BlockSpec((1, D), full),
            pl.BlockSpec((1, D), full),
            pl.BlockSpec((1, D), full),
        ],
        out_specs=pl.BlockSpec((BM, D), blk),
        out_shape=jax.ShapeDtypeStruct((B, D), jnp.float32),
        scratch_shapes=[pltpu.VMEM((D, D), jnp.float32),
                        pltpu.VMEM((1, D), jnp.float32)],
    )(p, cur_msg, WvT, WoT, bv2, bo2, g2, b2)


def kernel(idx, cur_msg, bank, Wq, bq, Wk, bk, Wv, bv, Wo, bo, gamma, beta):
    B, D = cur_msg.shape
    N, M, _ = bank.shape
    q2 = _stage_a(cur_msg, Wq.T, Wk, bq.reshape(1, D))
    p = _stage_b(idx, q2, bank.reshape(N, M * D))
    return _stage_c(p, cur_msg, Wv.T, Wo.T, bv.reshape(1, D),
                    bo.reshape(1, D), gamma.reshape(1, D), beta.reshape(1, D))


# speedup vs baseline: 1.2099x; 1.2099x over previous
import functools

import jax
import jax.numpy as jnp
from jax import lax
from jax.experimental import pallas as pl
from jax.experimental.pallas import tpu as pltpu
from jax.experimental.pallas import tpu_sc as plsc

_LANES = 16


def _bcast(v, t):
    dn = lax.GatherDimensionNumbers(
        offset_dims=(), collapsed_slice_dims=(0,), start_index_map=(0,))
    return lax.gather(v, jnp.full((_LANES, 1), t, jnp.int32), dn, (1,),
                      mode=lax.GatherScatterMode.PROMISE_IN_BOUNDS)


def _stage_a(cur_msg, WqT, Wk, bq2):
    B, D = cur_msg.shape
    BM = 256

    def body(msg_ref, WqT_ref, Wk_ref, bq2_ref, q2_ref, Wqk_s, bqk_s):
        @pl.when(pl.program_id(0) == 0)
        def _():
            Wqk_s[...] = jnp.dot(WqT_ref[...], Wk_ref[...],
                                 preferred_element_type=jnp.float32)
            bqk_s[...] = jnp.dot(bq2_ref[...], Wk_ref[...],
                                 preferred_element_type=jnp.float32)

        q2_ref[...] = jnp.dot(msg_ref[...], Wqk_s[...],
                              preferred_element_type=jnp.float32) + bqk_s[...]

    return pl.pallas_call(
        body,
        grid=(B // BM,),
        in_specs=[
            pl.BlockSpec((BM, D), lambda i: (i, 0)),
            pl.BlockSpec((D, D), lambda i: (0, 0)),
            pl.BlockSpec((D, D), lambda i: (0, 0)),
            pl.BlockSpec((1, D), lambda i: (0, 0)),
        ],
        out_specs=pl.BlockSpec((BM, D), lambda i: (i, 0)),
        out_shape=jax.ShapeDtypeStruct((B, D), jnp.float32),
        scratch_shapes=[pltpu.VMEM((D, D), jnp.float32),
                        pltpu.VMEM((1, D), jnp.float32)],
    )(cur_msg, WqT, Wk, bq2)


def _bcast_dyn(v, t):
    dn = lax.GatherDimensionNumbers(
        offset_dims=(), collapsed_slice_dims=(0,), start_index_map=(0,))
    return lax.gather(v, jnp.full((_LANES, 1), 1, jnp.int32) * t, dn, (1,),
                      mode=lax.GatherScatterMode.PROMISE_IN_BOUNDS)


def _stage_b(idx, q2, bank3, M):
    B, D = q2.shape
    NW = 32
    bw = B // NW
    CB = 4
    nch = bw // CB
    scale = D ** -0.5
    DC = D // _LANES

    mesh = plsc.VectorSubcoreMesh(core_axis_name="c", subcore_axis_name="s")

    @functools.partial(
        pl.kernel, mesh=mesh,
        out_type=jax.ShapeDtypeStruct((B, D), jnp.float32),
        compiler_params=pltpu.CompilerParams(needs_layout_passes=False),
        scratch_types=[
            pltpu.VMEM((bw,), jnp.int32),
            pltpu.VMEM((2, CB * M), jnp.int32),
            pltpu.VMEM((2, CB * M, D), jnp.float32),
            pltpu.VMEM((2, CB, D), jnp.float32),
            pltpu.VMEM((2, CB, D), jnp.float32),
            pltpu.SemaphoreType.DMA,
            pltpu.SemaphoreType.DMA,
        ])
    def sc(idx_hbm, q2_hbm, bank_hbm, p_hbm,
           idx_v, isub_v, rows_v, q2_v, p_v, sem0, sem1):
        wid = lax.axis_index("s") * 2 + lax.axis_index("c")
        base = wid * bw
        pltpu.sync_copy(idx_hbm.at[pl.ds(base, bw)], idx_v)
        lane = lax.broadcasted_iota(jnp.int32, (_LANES,), 0)
        sems = (sem0, sem1)

        def prep(c, k):
            idxc = idx_v[pl.ds((c // 4) * _LANES, _LANES)]
            for j in range(CB):
                bj = _bcast_dyn(idxc, (c % 4) * CB + j)
                isub_v[k, pl.ds(j * M, M)] = bj * M + lane
            pltpu.async_copy(bank_hbm.at[isub_v.at[k]], rows_v.at[k], sems[k])
            pltpu.async_copy(q2_hbm.at[pl.ds(base + c * CB, CB)],
                             q2_v.at[k], sems[k])

        def drain(k):
            pltpu.make_async_copy(
                bank_hbm.at[pl.ds(0, CB * M)], rows_v.at[k], sems[k]).wait()
            pltpu.make_async_copy(
                q2_hbm.at[pl.ds(0, CB)], q2_v.at[k], sems[k]).wait()

        def compute_b(k, j):
            jrow = j * M
            rowvec = (jrow + lane)

            def p1(dc, accs):
                q2c = q2_v[k, j, pl.ds(dc * _LANES, _LANES)]
                accs = list(accs)
                for t in range(_LANES):
                    qb = _bcast(q2c, t)
                    g = plsc.load_gather(
                        rows_v.at[k],
                        [rowvec, jnp.full((_LANES,), dc * _LANES + t,
                                          jnp.int32)])
                    accs[t % 8] = accs[t % 8] + g * qb
                return tuple(accs)

            accs = lax.fori_loop(
                0, DC, p1, tuple(jnp.zeros((_LANES,), jnp.float32)
                                 for _ in range(8)))
            s = (((accs[0] + accs[1]) + (accs[2] + accs[3]))
                 + ((accs[4] + accs[5]) + (accs[6] + accs[7]))) * scale
            e = jnp.exp(s - jnp.max(s))
            a = e / jnp.sum(e)
            ab = [_bcast(a, m) for m in range(M)]

            def p2(dc, carry):
                acc = [jnp.zeros((_LANES,), jnp.float32) for _ in range(4)]
                for m in range(M):
                    acc[m % 4] = acc[m % 4] + (
                        rows_v[k, jrow + m, pl.ds(dc * _LANES, _LANES)]
                        * ab[m])
                p_v[k, j, pl.ds(dc * _LANES, _LANES)] = (
                    (acc[0] + acc[1]) + (acc[2] + acc[3]))
                return carry

            lax.fori_loop(0, DC, p2, 0)

        prep(0, 0)

        def pair(c2, carry):
            c0 = c2 * 2
            for k in (0, 1):
                c = c0 + k

                @pl.when(c + 1 < nch)
                def _():
                    prep(c + 1, 1 - k)

                drain(k)
                for j in range(CB):
                    compute_b(k, j)
                pltpu.sync_copy(p_v.at[k],
                                p_hbm.at[pl.ds(base + c * CB, CB)])
            return carry

        lax.fori_loop(0, nch // 2, pair, 0)

    return sc(idx, q2, bank3)


def _stage_c(p, cur_msg, WvT, WoT, bv2, bo2, g2, b2):
    B, D = cur_msg.shape
    BM = 256

    def body(p_ref, msg_ref, WvT_ref, WoT_ref, bv2_ref, bo2_ref,
             g_ref, b_ref, o_ref, Wvo_s, bvo_s):
        @pl.when(pl.program_id(0) == 0)
        def _():
            Wvo_s[...] = jnp.dot(WvT_ref[...], WoT_ref[...],
                                 preferred_element_type=jnp.float32)
            bvo_s[...] = jnp.dot(bv2_ref[...], WoT_ref[...],
                                 preferred_element_type=jnp.float32) + bo2_ref[...]

        x = msg_ref[...] + jnp.dot(p_ref[...], Wvo_s[...],
                                   preferred_element_type=jnp.float32) + bvo_s[...]
        mu = jnp.mean(x, axis=1, keepdims=True)
        xc = x - mu
        var = jnp.mean(xc * xc, axis=1, keepdims=True)
        o_ref[...] = xc * lax.rsqrt(var + 1e-5) * g_ref[...] + b_ref[...]

    full = lambda i: (0, 0)
    blk = lambda i: (i, 0)
    return pl.pallas_call(
        body,
        grid=(B // BM,),
        in_specs=[
            pl.BlockSpec((BM, D), blk),
            pl.BlockSpec((BM, D), blk),
            pl.BlockSpec((D, D), full),
            pl.BlockSpec((D, D), full),
            pl.BlockSpec((1, D), full),
            pl.BlockSpec((1, D), full),
            pl.BlockSpec((1, D), full),
            pl.BlockSpec((1, D), full),
        ],
        out_specs=pl.BlockSpec((BM, D), blk),
        out_shape=jax.ShapeDtypeStruct((B, D), jnp.float32),
        scratch_shapes=[pltpu.VMEM((D, D), jnp.float32),
                        pltpu.VMEM((1, D), jnp.float32)],
    )(p, cur_msg, WvT, WoT, bv2, bo2, g2, b2)


def kernel(idx, cur_msg, bank, Wq, bq, Wk, bk, Wv, bv, Wo, bo, gamma, beta):
    B, D = cur_msg.shape
    N, M, _ = bank.shape
    q2 = _stage_a(cur_msg, Wq.T, Wk, bq.reshape(1, D))
    p = _stage_b(idx, q2, bank.reshape(N * M, D), M)
    return _stage_c(p, cur_msg, Wv.T, Wo.T, bv.reshape(1, D),
                    bo.reshape(1, D), gamma.reshape(1, D), beta.reshape(1, D))


# speedup vs baseline: 4.5421x; 3.7540x over previous
import functools

import jax
import jax.numpy as jnp
from jax import lax
from jax.experimental import pallas as pl
from jax.experimental.pallas import tpu as pltpu
from jax.experimental.pallas import tpu_sc as plsc

_LANES = 16


def _bcast(v, t):
    dn = lax.GatherDimensionNumbers(
        offset_dims=(), collapsed_slice_dims=(0,), start_index_map=(0,))
    return lax.gather(v, jnp.full((_LANES, 1), t, jnp.int32), dn, (1,),
                      mode=lax.GatherScatterMode.PROMISE_IN_BOUNDS)


def _stage_a(cur_msg, WqT, Wk, bq2):
    B, D = cur_msg.shape
    BM = 256

    def body(msg_ref, WqT_ref, Wk_ref, bq2_ref, q2_ref, Wqk_s, bqk_s):
        @pl.when(pl.program_id(0) == 0)
        def _():
            Wqk_s[...] = jnp.dot(WqT_ref[...], Wk_ref[...],
                                 preferred_element_type=jnp.float32)
            bqk_s[...] = jnp.dot(bq2_ref[...], Wk_ref[...],
                                 preferred_element_type=jnp.float32)

        q2_ref[...] = jnp.dot(msg_ref[...], Wqk_s[...],
                              preferred_element_type=jnp.float32) + bqk_s[...]

    return pl.pallas_call(
        body,
        grid=(B // BM,),
        in_specs=[
            pl.BlockSpec((BM, D), lambda i: (i, 0)),
            pl.BlockSpec((D, D), lambda i: (0, 0)),
            pl.BlockSpec((D, D), lambda i: (0, 0)),
            pl.BlockSpec((1, D), lambda i: (0, 0)),
        ],
        out_specs=pl.BlockSpec((BM, D), lambda i: (i, 0)),
        out_shape=jax.ShapeDtypeStruct((B, D), jnp.float32),
        scratch_shapes=[pltpu.VMEM((D, D), jnp.float32),
                        pltpu.VMEM((1, D), jnp.float32)],
    )(cur_msg, WqT, Wk, bq2)


def _bcast_dyn(v, t):
    dn = lax.GatherDimensionNumbers(
        offset_dims=(), collapsed_slice_dims=(0,), start_index_map=(0,))
    return lax.gather(v, jnp.full((_LANES, 1), 1, jnp.int32) * t, dn, (1,),
                      mode=lax.GatherScatterMode.PROMISE_IN_BOUNDS)


def _stage_b(idx, q2, bank3, M):
    B, D = q2.shape
    NW = 32
    bw = B // NW
    CB = 4
    nch = bw // CB
    scale = D ** -0.5
    DC = D // _LANES

    mesh = plsc.VectorSubcoreMesh(core_axis_name="c", subcore_axis_name="s")

    @functools.partial(
        pl.kernel, mesh=mesh,
        out_type=jax.ShapeDtypeStruct((B, D), jnp.float32),
        compiler_params=pltpu.CompilerParams(needs_layout_passes=False),
        scratch_types=[
            pltpu.VMEM((bw,), jnp.int32),
            pltpu.VMEM((2, CB * M), jnp.int32),
            pltpu.VMEM((2, CB * M, D), jnp.float32),
            pltpu.VMEM((2, CB, D), jnp.float32),
            pltpu.VMEM((2, CB, D), jnp.float32),
            pltpu.SemaphoreType.DMA,
            pltpu.SemaphoreType.DMA,
        ])
    def sc(idx_hbm, q2_hbm, bank_hbm, p_hbm,
           idx_v, isub_v, rows_v, q2_v, p_v, sem0, sem1):
        wid = lax.axis_index("s") * 2 + lax.axis_index("c")
        base = wid * bw
        pltpu.sync_copy(idx_hbm.at[pl.ds(base, bw)], idx_v)
        lane = lax.broadcasted_iota(jnp.int32, (_LANES,), 0)
        sems = (sem0, sem1)

        def prep(c, k):
            idxc = idx_v[pl.ds((c // 4) * _LANES, _LANES)]
            for j in range(CB):
                bj = _bcast_dyn(idxc, (c % 4) * CB + j)
                isub_v[k, pl.ds(j * M, M)] = bj * M + lane
            pltpu.async_copy(bank_hbm.at[isub_v.at[k]], rows_v.at[k], sems[k])
            pltpu.async_copy(q2_hbm.at[pl.ds(base + c * CB, CB)],
                             q2_v.at[k], sems[k])

        def drain(k):
            pltpu.make_async_copy(
                bank_hbm.at[pl.ds(0, CB * M)], rows_v.at[k], sems[k]).wait()
            pltpu.make_async_copy(
                q2_hbm.at[pl.ds(0, CB)], q2_v.at[k], sems[k]).wait()

        def compute_b(k, j):
            jrow = j * M

            def p1(dc, accs):
                q2c = q2_v[k, j, pl.ds(dc * _LANES, _LANES)]
                accs = list(accs)
                for m in range(M):
                    g = rows_v[k, jrow + m, pl.ds(dc * _LANES, _LANES)]
                    accs[m] = accs[m] + g * q2c
                return tuple(accs)

            accs = lax.fori_loop(
                0, DC, p1, tuple(jnp.zeros((_LANES,), jnp.float32)
                                 for _ in range(M)))
            s = jnp.zeros((_LANES,), jnp.float32)
            for m in range(M):
                s = jnp.where(lane == m, jnp.sum(accs[m]), s)
            s = s * scale
            e = jnp.exp(s - jnp.max(s))
            a = e / jnp.sum(e)
            ab = [_bcast(a, m) for m in range(M)]

            def p2(dc, carry):
                acc = [jnp.zeros((_LANES,), jnp.float32) for _ in range(4)]
                for m in range(M):
                    acc[m % 4] = acc[m % 4] + (
                        rows_v[k, jrow + m, pl.ds(dc * _LANES, _LANES)]
                        * ab[m])
                p_v[k, j, pl.ds(dc * _LANES, _LANES)] = (
                    (acc[0] + acc[1]) + (acc[2] + acc[3]))
                return carry

            lax.fori_loop(0, DC, p2, 0)

        prep(0, 0)

        def pair(c2, carry):
            c0 = c2 * 2
            for k in (0, 1):
                c = c0 + k

                @pl.when(c + 1 < nch)
                def _():
                    prep(c + 1, 1 - k)

                drain(k)
                for j in range(CB):
                    compute_b(k, j)
                pltpu.sync_copy(p_v.at[k],
                                p_hbm.at[pl.ds(base + c * CB, CB)])
            return carry

        lax.fori_loop(0, nch // 2, pair, 0)

    return sc(idx, q2, bank3)


def _stage_c(p, cur_msg, WvT, WoT, bv2, bo2, g2, b2):
    B, D = cur_msg.shape
    BM = 256

    def body(p_ref, msg_ref, WvT_ref, WoT_ref, bv2_ref, bo2_ref,
             g_ref, b_ref, o_ref, Wvo_s, bvo_s):
        @pl.when(pl.program_id(0) == 0)
        def _():
            Wvo_s[...] = jnp.dot(WvT_ref[...], WoT_ref[...],
                                 preferred_element_type=jnp.float32)
            bvo_s[...] = jnp.dot(bv2_ref[...], WoT_ref[...],
                                 preferred_element_type=jnp.float32) + bo2_ref[...]

        x = msg_ref[...] + jnp.dot(p_ref[...], Wvo_s[...],
                                   preferred_element_type=jnp.float32) + bvo_s[...]
        mu = jnp.mean(x, axis=1, keepdims=True)
        xc = x - mu
        var = jnp.mean(xc * xc, axis=1, keepdims=True)
        o_ref[...] = xc * lax.rsqrt(var + 1e-5) * g_ref[...] + b_ref[...]

    full = lambda i: (0, 0)
    blk = lambda i: (i, 0)
    return pl.pallas_call(
        body,
        grid=(B // BM,),
        in_specs=[
            pl.BlockSpec((BM, D), blk),
            pl.BlockSpec((BM, D), blk),
            pl.BlockSpec((D, D), full),
            pl.BlockSpec((D, D), full),
            pl.BlockSpec((1, D), full),
            pl.BlockSpec((1, D), full),
            pl.BlockSpec((1, D), full),
            pl.BlockSpec((1, D), full),
        ],
        out_specs=pl.BlockSpec((BM, D), blk),
        out_shape=jax.ShapeDtypeStruct((B, D), jnp.float32),
        scratch_shapes=[pltpu.VMEM((D, D), jnp.float32),
                        pltpu.VMEM((1, D), jnp.float32)],
    )(p, cur_msg, WvT, WoT, bv2, bo2, g2, b2)


def kernel(idx, cur_msg, bank, Wq, bq, Wk, bk, Wv, bv, Wo, bo, gamma, beta):
    B, D = cur_msg.shape
    N, M, _ = bank.shape
    q2 = _stage_a(cur_msg, Wq.T, Wk, bq.reshape(1, D))
    p = _stage_b(idx, q2, bank.reshape(N * M, D), M)
    return _stage_c(p, cur_msg, Wv.T, Wo.T, bv.reshape(1, D),
                    bo.reshape(1, D), gamma.reshape(1, D), beta.reshape(1, D))
